# baseline (speedup 1.0000x reference)
import functools

import numpy as np

import jax
import jax.numpy as jnp
from jax.experimental import pallas as pl
from jax.experimental.pallas import tpu as pltpu
from jax.experimental.shard_map import shard_map
from jax.sharding import Mesh, PartitionSpec as P

_BM = 200


def _prop1_kernel(adj_ref, x_ref, w_ref, b_ref, o_ref, s_ref):
    @pl.when(pl.program_id(0) == 0)
    def _():
        s_ref[...] = jnp.dot(x_ref[...], w_ref[...],
                             preferred_element_type=jnp.float32)

    acc = jnp.dot(adj_ref[...], s_ref[...],
                  preferred_element_type=jnp.float32)
    o_ref[...] = jnp.maximum(acc + b_ref[...], 0.0)


def _mm_kernel(a_ref, w_ref, o_ref):
    o_ref[...] = jnp.dot(a_ref[...], w_ref[...],
                         preferred_element_type=jnp.float32)


def _prop2_kernel(adj_ref, s_ref, b_ref, mu_ref, lv_ref):
    acc = jnp.dot(adj_ref[...], s_ref[...],
                  preferred_element_type=jnp.float32)
    acc = jnp.maximum(acc + b_ref[...], 0.0)
    mu_ref[...] = acc[:, :32]
    lv_ref[...] = acc[:, 32:]


def _dec_kernel(za_ref, zb_ref, o_ref, zb_bf_ref):
    @pl.when(pl.program_id(0) == 0)
    def _():
        zb_bf_ref[...] = zb_ref[...].astype(jnp.bfloat16)

    prod = jax.lax.dot_general(za_ref[...].astype(jnp.bfloat16),
                               zb_bf_ref[...],
                               (((1,), (1,)), ((), ())),
                               preferred_element_type=jnp.float32)
    o_ref[...] = jax.nn.sigmoid(prod)


def _local_forward(x, adj_l, W0, b0, W12, b12):
    m, n = adj_l.shape
    nfeat = x.shape[1]
    nhid = W0.shape[1]
    zdim = W12.shape[1] // 2
    grid = (m // _BM,)
    seq = pltpu.CompilerParams(dimension_semantics=("arbitrary",))

    h_l = pl.pallas_call(
        _prop1_kernel,
        grid=grid,
        in_specs=[
            pl.BlockSpec((_BM, n), lambda i: (i, 0)),
            pl.BlockSpec((n, nfeat), lambda i: (0, 0)),
            pl.BlockSpec((nfeat, nhid), lambda i: (0, 0)),
            pl.BlockSpec((1, nhid), lambda i: (0, 0)),
        ],
        out_specs=pl.BlockSpec((_BM, nhid), lambda i: (i, 0)),
        out_shape=jax.ShapeDtypeStruct((m, nhid), jnp.float32),
        scratch_shapes=[pltpu.VMEM((n, nhid), jnp.float32)],
        compiler_params=seq,
    )(adj_l, x, W0, b0)

    s12_l = pl.pallas_call(
        _mm_kernel,
        out_shape=jax.ShapeDtypeStruct((m, 2 * zdim), jnp.float32),
    )(h_l, W12)
    s12 = jax.lax.all_gather(s12_l, "x", axis=0, tiled=True)

    mu_l, lv_l = pl.pallas_call(
        _prop2_kernel,
        grid=grid,
        in_specs=[
            pl.BlockSpec((_BM, n), lambda i: (i, 0)),
            pl.BlockSpec((n, 2 * zdim), lambda i: (0, 0)),
            pl.BlockSpec((1, 2 * zdim), lambda i: (0, 0)),
        ],
        out_specs=[
            pl.BlockSpec((_BM, zdim), lambda i: (i, 0)),
            pl.BlockSpec((_BM, zdim), lambda i: (i, 0)),
        ],
        out_shape=[
            jax.ShapeDtypeStruct((m, zdim), jnp.float32),
            jax.ShapeDtypeStruct((m, zdim), jnp.float32),
        ],
        compiler_params=seq,
    )(adj_l, s12, b12)

    mu = jax.lax.all_gather(mu_l, "x", axis=0, tiled=True)

    rec_l = pl.pallas_call(
        _dec_kernel,
        grid=grid,
        in_specs=[
            pl.BlockSpec((_BM, zdim), lambda i: (i, 0)),
            pl.BlockSpec((n, zdim), lambda i: (0, 0)),
        ],
        out_specs=pl.BlockSpec((_BM, n), lambda i: (i, 0)),
        out_shape=jax.ShapeDtypeStruct((m, n), jnp.float32),
        scratch_shapes=[pltpu.VMEM((n, zdim), jnp.bfloat16)],
        compiler_params=seq,
    )(mu_l, mu)

    return rec_l, mu_l, lv_l


def kernel(x, adj, W0, b0, W1, b1, W2, b2):
    n = adj.shape[0]
    W12 = jnp.concatenate([W1, W2], axis=1)
    b12 = jnp.concatenate([b1, b2])[None, :]
    b0r = b0[None, :]

    devs = jax.devices()
    ndev = len(devs)
    while ndev > 1 and n % (ndev * _BM) != 0:
        ndev -= 1
    mesh = Mesh(np.array(devs[:ndev]), ("x",))

    fwd = shard_map(
        _local_forward,
        mesh=mesh,
        in_specs=(P(None, None), P("x", None), P(None, None), P(None, None),
                  P(None, None), P(None, None)),
        out_specs=(P("x", None), P("x", None), P("x", None)),
        check_rep=False,
    )
    adj_recon, mu, logvar = fwd(x, adj, W0, b0r, W12, b12)
    return (adj_recon, mu, mu, logvar)


# speedup vs baseline: 1.0884x; 1.0884x over previous
import jax
import jax.numpy as jnp
from jax.experimental import pallas as pl
from jax.experimental.pallas import tpu as pltpu


def _prop1_kernel(adj_ref, x_ref, w_ref, b_ref, o_ref, s_ref):
    @pl.when(pl.program_id(0) == 0)
    def _():
        s_ref[...] = jnp.dot(x_ref[...], w_ref[...],
                             preferred_element_type=jnp.float32)

    acc = jnp.dot(adj_ref[...], s_ref[...],
                  preferred_element_type=jnp.float32)
    o_ref[...] = jnp.maximum(acc + b_ref[...], 0.0)


def _prop2_kernel(adj_ref, h_ref, w_ref, b_ref, mu_ref, lv_ref, s_ref):
    @pl.when(pl.program_id(0) == 0)
    def _():
        s_ref[...] = jnp.dot(h_ref[...], w_ref[...],
                             preferred_element_type=jnp.float32)

    acc = jnp.dot(adj_ref[...], s_ref[...],
                  preferred_element_type=jnp.float32)
    acc = jnp.maximum(acc + b_ref[...], 0.0)
    mu_ref[...] = acc[:, :32]
    lv_ref[...] = acc[:, 32:]


def _dec_kernel(za_ref, zb_ref, o_ref, zb_bf_ref):
    @pl.when(pl.program_id(0) == 0)
    def _():
        zb_bf_ref[...] = zb_ref[...].astype(jnp.bfloat16)

    p = jax.lax.dot_general(za_ref[...].astype(jnp.bfloat16), zb_bf_ref[...],
                            (((1,), (1,)), ((), ())),
                            preferred_element_type=jnp.float32)
    o_ref[...] = jax.nn.sigmoid(p)


_BM = 400


def kernel(x, adj, W0, b0, W1, b1, W2, b2):
    n, nfeat = x.shape
    nhid = W0.shape[1]
    zdim = W1.shape[1]
    W12 = jnp.concatenate([W1, W2], axis=1)
    b12 = jnp.concatenate([b1, b2])[None, :]
    grid = (n // _BM,)
    seq = pltpu.CompilerParams(dimension_semantics=("arbitrary",))

    h = pl.pallas_call(
        _prop1_kernel,
        grid=grid,
        in_specs=[
            pl.BlockSpec((_BM, n), lambda i: (i, 0)),
            pl.BlockSpec((n, nfeat), lambda i: (0, 0)),
            pl.BlockSpec((nfeat, nhid), lambda i: (0, 0)),
            pl.BlockSpec((1, nhid), lambda i: (0, 0)),
        ],
        out_specs=pl.BlockSpec((_BM, nhid), lambda i: (i, 0)),
        out_shape=jax.ShapeDtypeStruct((n, nhid), jnp.float32),
        scratch_shapes=[pltpu.VMEM((n, nhid), jnp.float32)],
        compiler_params=seq,
    )(adj, x, W0, b0[None, :])

    mu, logvar = pl.pallas_call(
        _prop2_kernel,
        grid=grid,
        in_specs=[
            pl.BlockSpec((_BM, n), lambda i: (i, 0)),
            pl.BlockSpec((n, nhid), lambda i: (0, 0)),
            pl.BlockSpec((nhid, 2 * zdim), lambda i: (0, 0)),
            pl.BlockSpec((1, 2 * zdim), lambda i: (0, 0)),
        ],
        out_specs=[
            pl.BlockSpec((_BM, zdim), lambda i: (i, 0)),
            pl.BlockSpec((_BM, zdim), lambda i: (i, 0)),
        ],
        out_shape=[
            jax.ShapeDtypeStruct((n, zdim), jnp.float32),
            jax.ShapeDtypeStruct((n, zdim), jnp.float32),
        ],
        scratch_shapes=[pltpu.VMEM((n, 2 * zdim), jnp.float32)],
        compiler_params=seq,
    )(adj, h, W12, b12)

    adj_recon = pl.pallas_call(
        _dec_kernel,
        grid=grid,
        in_specs=[
            pl.BlockSpec((_BM, zdim), lambda i: (i, 0)),
            pl.BlockSpec((n, zdim), lambda i: (0, 0)),
        ],
        out_specs=pl.BlockSpec((_BM, n), lambda i: (i, 0)),
        out_shape=jax.ShapeDtypeStruct((n, n), jnp.float32),
        scratch_shapes=[pltpu.VMEM((n, zdim), jnp.bfloat16)],
        compiler_params=seq,
    )(mu, mu)

    return (adj_recon, mu, mu, logvar)
